# baseline (speedup 1.0000x reference)
import functools

import jax
import jax.numpy as jnp
from jax import lax
from jax.experimental import pallas as pl
from jax.experimental.pallas import tpu as pltpu
from jax.experimental.pallas import tpu_sc as plsc

N_WORD = 100000
N_DIM = 64
CONTEXT = 4
HIDDEN = 128
IN_DIM = 2 * CONTEXT * N_DIM

NB = 25
R = N_WORD // NB


def _sc_gather(xi, emb):
    mesh = plsc.VectorSubcoreMesh(core_axis_name="c", subcore_axis_name="s")

    @functools.partial(
        pl.kernel,
        out_type=jax.ShapeDtypeStruct((2 * CONTEXT, N_DIM), jnp.float32),
        mesh=mesh,
        scratch_types=[
            pltpu.VMEM((2 * CONTEXT,), jnp.int32),
            pltpu.VMEM((2 * CONTEXT, N_DIM), jnp.float32),
            pltpu.SemaphoreType.DMA,
        ],
        compiler_params=pltpu.CompilerParams(use_tc_tiling_on_sc=False),
    )
    def gather_kernel(idx_hbm, table_hbm, out_hbm, idx_v, rows_v, sem):
        wid = lax.axis_index("s") * 2 + lax.axis_index("c")

        @pl.when(wid == 0)
        def _():
            pltpu.sync_copy(idx_hbm, idx_v)
            pltpu.async_copy(table_hbm.at[idx_v], rows_v, sem).wait()
            pltpu.sync_copy(rows_v, out_hbm)

    return gather_kernel(xi, emb)


def _mlp_body(hcat_ref, w1_ref, b1_ref, w2_ref, b2_ref, out_ref, hrel_ref, m_ref):
    j = pl.program_id(0)

    @pl.when(j == 0)
    def _():
        h1 = lax.dot_general(
            hcat_ref[...], w1_ref[...], (((1,), (1,)), ((), ())),
            preferred_element_type=jnp.float32)
        hrel_ref[...] = jnp.maximum(h1 + b1_ref[...], 0.0)
        m_ref[0] = -jnp.inf

    @pl.when(j < NB)
    def _():
        logits = lax.dot_general(
            hrel_ref[...], w2_ref[...], (((1,), (1,)), ((), ())),
            preferred_element_type=jnp.float32) + b2_ref[0]
        out_ref[pl.ds(j, 1), :] = logits
        m_ref[0] = jnp.maximum(m_ref[0], jnp.max(logits))

    @pl.when(j == NB)
    def _():
        logits = out_ref[...]
        m = m_ref[0]
        s = jnp.sum(jnp.exp(logits - m))
        out_ref[...] = logits - (m + jnp.log(s))


def _mlp(hcat, w1, b1r, w2, b2r):
    return pl.pallas_call(
        _mlp_body,
        grid=(NB + 1,),
        in_specs=[
            pl.BlockSpec((1, IN_DIM), lambda j: (0, 0)),
            pl.BlockSpec((HIDDEN, IN_DIM), lambda j: (0, 0)),
            pl.BlockSpec((1, HIDDEN), lambda j: (0, 0)),
            pl.BlockSpec((R, HIDDEN), lambda j: (jnp.minimum(j, NB - 1), 0)),
            pl.BlockSpec((1, 1, R), lambda j: (jnp.minimum(j, NB - 1), 0, 0)),
        ],
        out_specs=pl.BlockSpec((NB, R), lambda j: (0, 0)),
        out_shape=jax.ShapeDtypeStruct((NB, R), jnp.float32),
        scratch_shapes=[
            pltpu.VMEM((1, HIDDEN), jnp.float32),
            pltpu.SMEM((1,), jnp.float32),
        ],
        compiler_params=pltpu.CompilerParams(
            dimension_semantics=("arbitrary",)),
    )(hcat, w1, b1r, w2, b2r)


@jax.jit
def kernel(x, emb, w1, b1, w2, b2):
    rows = _sc_gather(x.astype(jnp.int32), emb)
    hcat = rows.reshape(1, IN_DIM)
    out = _mlp(hcat, w1, b1.reshape(1, HIDDEN), w2, b2.reshape(NB, 1, R))
    return out.reshape(1, N_WORD)


# speedup vs baseline: 1.5719x; 1.5719x over previous
import jax
import jax.numpy as jnp
from jax import lax
from jax.experimental import pallas as pl
from jax.experimental.pallas import tpu as pltpu

N_WORD = 100000
N_DIM = 64
CONTEXT = 4
NCTX = 2 * CONTEXT
HIDDEN = 128
IN_DIM = NCTX * N_DIM

NB = 25
R = N_WORD // NB


def _body(x_ref, *refs):
    emb_refs = refs[:NCTX]
    w1_ref, b1_ref, w2_ref, b2_ref, out_ref, hrel_ref, m_ref = refs[NCTX:]
    j = pl.program_id(0)

    @pl.when(j == 0)
    def _():
        h1 = b1_ref[...]
        for i in range(NCTX):
            row = emb_refs[i][pl.ds(x_ref[i] % 8, 1), :]
            h1 = h1 + lax.dot_general(
                row, w1_ref[:, i * N_DIM:(i + 1) * N_DIM],
                (((1,), (1,)), ((), ())),
                preferred_element_type=jnp.float32)
        hrel_ref[...] = jnp.maximum(h1, 0.0)
        m_ref[0] = -jnp.inf

    @pl.when(j < NB)
    def _():
        logits = lax.dot_general(
            hrel_ref[...], w2_ref[...], (((1,), (1,)), ((), ())),
            preferred_element_type=jnp.float32) + b2_ref[0]
        out_ref[pl.ds(j, 1), :] = logits
        m_ref[0] = jnp.maximum(m_ref[0], jnp.max(logits))

    @pl.when(j == NB)
    def _():
        logits = out_ref[...]
        m = m_ref[0]
        s = jnp.sum(jnp.exp(logits - m))
        out_ref[...] = logits - (m + jnp.log(s))


@jax.jit
def kernel(x, emb, w1, b1, w2, b2):
    xi = x.astype(jnp.int32)
    emb_spec = [
        pl.BlockSpec((8, N_DIM), lambda j, xr, i=i: (xr[i] // 8, 0))
        for i in range(NCTX)
    ]
    out = pl.pallas_call(
        _body,
        grid_spec=pltpu.PrefetchScalarGridSpec(
            num_scalar_prefetch=1,
            grid=(NB + 1,),
            in_specs=emb_spec + [
                pl.BlockSpec((HIDDEN, IN_DIM), lambda j, xr: (0, 0)),
                pl.BlockSpec((1, HIDDEN), lambda j, xr: (0, 0)),
                pl.BlockSpec((R, HIDDEN), lambda j, xr: (jnp.minimum(j, NB - 1), 0)),
                pl.BlockSpec((1, 1, R), lambda j, xr: (jnp.minimum(j, NB - 1), 0, 0)),
            ],
            out_specs=pl.BlockSpec((NB, R), lambda j, xr: (0, 0)),
            scratch_shapes=[
                pltpu.VMEM((1, HIDDEN), jnp.float32),
                pltpu.SMEM((1,), jnp.float32),
            ],
        ),
        out_shape=jax.ShapeDtypeStruct((NB, R), jnp.float32),
        compiler_params=pltpu.CompilerParams(
            dimension_semantics=("arbitrary",)),
    )(xi, *([emb] * NCTX), w1, b1.reshape(1, HIDDEN), w2, b2.reshape(NB, 1, R))
    return out.reshape(1, N_WORD)


# speedup vs baseline: 1.7141x; 1.0905x over previous
import jax
import jax.numpy as jnp
from jax import lax
from jax.experimental import pallas as pl
from jax.experimental.pallas import tpu as pltpu

N_WORD = 100000
N_DIM = 64
CONTEXT = 4
NCTX = 2 * CONTEXT
HIDDEN = 128
IN_DIM = NCTX * N_DIM

NB = 25
R = N_WORD // NB


def _body(x_ref, *refs):
    emb_refs = refs[:NCTX]
    w1_ref, b1_ref, w2_ref, b2_ref, out_ref, hrel_ref, sc_ref = refs[NCTX:]
    j = pl.program_id(0)

    @pl.when(j == 0)
    def _():
        h1 = b1_ref[...].reshape(1, HIDDEN)
        for i in range(NCTX):
            row = emb_refs[i][pl.ds(x_ref[i] % 8, 1), :]
            h1 = h1 + lax.dot_general(
                row, w1_ref[:, i * N_DIM:(i + 1) * N_DIM],
                (((1,), (1,)), ((), ())),
                preferred_element_type=jnp.float32)
        hrel_ref[...] = jnp.maximum(h1, 0.0)

    @pl.when(j < NB)
    def _():
        logits = lax.dot_general(
            hrel_ref[...], w2_ref[...], (((1,), (1,)), ((), ())),
            preferred_element_type=jnp.float32)
        sc_ref[pl.ds(j, 1), :] = logits

    @pl.when(j == NB)
    def _():
        for jj in range(NB):
            sc_ref[jj:jj + 1, :] = (
                sc_ref[jj:jj + 1, :]
                + b2_ref[pl.ds(jj * R, R)].reshape(1, R))
        h2 = sc_ref[...]
        m = jnp.max(h2)
        norm = m + jnp.log(jnp.sum(jnp.exp(h2 - m)))
        res = h2 - norm
        for jj in range(NB):
            out_ref[0:1, jj * R:(jj + 1) * R] = res[jj:jj + 1, :]


@jax.jit
def kernel(x, emb, w1, b1, w2, b2):
    xi = x.astype(jnp.int32)
    emb_spec = [
        pl.BlockSpec((8, N_DIM), lambda j, xr, i=i: (xr[i] // 8, 0))
        for i in range(NCTX)
    ]
    return pl.pallas_call(
        _body,
        grid_spec=pltpu.PrefetchScalarGridSpec(
            num_scalar_prefetch=1,
            grid=(NB + 1,),
            in_specs=emb_spec + [
                pl.BlockSpec((HIDDEN, IN_DIM), lambda j, xr: (0, 0)),
                pl.BlockSpec(memory_space=pltpu.VMEM),
                pl.BlockSpec((R, HIDDEN), lambda j, xr: (jnp.minimum(j, NB - 1), 0)),
                pl.BlockSpec(memory_space=pltpu.VMEM),
            ],
            out_specs=pl.BlockSpec((1, N_WORD), lambda j, xr: (0, 0)),
            scratch_shapes=[
                pltpu.VMEM((1, HIDDEN), jnp.float32),
                pltpu.VMEM((NB, R), jnp.float32),
            ],
        ),
        out_shape=jax.ShapeDtypeStruct((1, N_WORD), jnp.float32),
        compiler_params=pltpu.CompilerParams(
            dimension_semantics=("arbitrary",)),
    )(xi, *([emb] * NCTX), w1, b1, w2, b2)


# speedup vs baseline: 3.6820x; 2.1481x over previous
import jax
import jax.numpy as jnp
from jax import lax
from jax.experimental import pallas as pl
from jax.experimental.pallas import tpu as pltpu

N_WORD = 100000
N_DIM = 64
CONTEXT = 4
NCTX = 2 * CONTEXT
HIDDEN = 128
IN_DIM = NCTX * N_DIM

NB = 25
R = N_WORD // NB


def _body(x_ref, *refs):
    emb_refs = refs[:NCTX]
    w1_ref, b1_ref, w2_ref, b2_ref, out_ref, hrel_ref, sc_ref = refs[NCTX:]
    j = pl.program_id(0)

    @pl.when(j == 0)
    def _():
        h1 = b1_ref[...].reshape(1, HIDDEN)
        lane = lax.broadcasted_iota(jnp.int32, (1, 128), 1)
        for i in range(NCTX):
            onehot = (lane == x_ref[i] % 128).astype(jnp.float32)
            col = lax.dot_general(
                onehot, emb_refs[i][...], (((1,), (1,)), ((), ())),
                preferred_element_type=jnp.float32)
            h1 = h1 + lax.dot_general(
                col, w1_ref[:, i * N_DIM:(i + 1) * N_DIM],
                (((1,), (1,)), ((), ())),
                preferred_element_type=jnp.float32)
        hrel_ref[...] = jnp.maximum(h1, 0.0)

    @pl.when(j < NB)
    def _():
        logits = lax.dot_general(
            hrel_ref[...], w2_ref[...], (((1,), (1,)), ((), ())),
            preferred_element_type=jnp.float32)
        sc_ref[pl.ds(j, 1), :] = logits

    @pl.when(j == NB)
    def _():
        for jj in range(NB):
            sc_ref[jj:jj + 1, :] = (
                sc_ref[jj:jj + 1, :]
                + b2_ref[pl.ds(jj * R, R)].reshape(1, R))
        h2 = sc_ref[...]
        m = jnp.max(h2)
        norm = m + jnp.log(jnp.sum(jnp.exp(h2 - m)))
        res = h2 - norm
        for jj in range(NB):
            out_ref[0:1, jj * R:(jj + 1) * R] = res[jj:jj + 1, :]


@jax.jit
def kernel(x, emb, w1, b1, w2, b2):
    xi = x.astype(jnp.int32)
    embt = emb.T
    emb_spec = [
        pl.BlockSpec((N_DIM, 128), lambda j, xr, i=i: (0, xr[i] // 128))
        for i in range(NCTX)
    ]
    return pl.pallas_call(
        _body,
        grid_spec=pltpu.PrefetchScalarGridSpec(
            num_scalar_prefetch=1,
            grid=(NB + 1,),
            in_specs=emb_spec + [
                pl.BlockSpec((HIDDEN, IN_DIM), lambda j, xr: (0, 0)),
                pl.BlockSpec(memory_space=pltpu.VMEM),
                pl.BlockSpec((R, HIDDEN), lambda j, xr: (jnp.minimum(j, NB - 1), 0)),
                pl.BlockSpec(memory_space=pltpu.VMEM),
            ],
            out_specs=pl.BlockSpec((1, N_WORD), lambda j, xr: (0, 0)),
            scratch_shapes=[
                pltpu.VMEM((1, HIDDEN), jnp.float32),
                pltpu.VMEM((NB, R), jnp.float32),
            ],
        ),
        out_shape=jax.ShapeDtypeStruct((1, N_WORD), jnp.float32),
        compiler_params=pltpu.CompilerParams(
            dimension_semantics=("arbitrary",)),
    )(xi, *([embt] * NCTX), w1, b1, w2, b2)
